# baseline (speedup 1.0000x reference)
import functools

import jax
import jax.numpy as jnp
from jax import lax
from jax.experimental import pallas as pl
from jax.experimental.pallas import tpu as pltpu
from jax.experimental.pallas import tpu_sc as plsc

_NCHUNK = 4
_C = 128


def _make_gather(V, D, B):
  info = plsc.get_sparse_core_info()
  NW = info.num_cores * info.num_subcores
  b_per_w = B // NW
  assert b_per_w == _NCHUNK * _C and D == 64
  mesh = plsc.VectorSubcoreMesh(core_axis_name="c", subcore_axis_name="s")

  @functools.partial(
      pl.kernel,
      out_type=jax.ShapeDtypeStruct((D, B), jnp.float32),
      mesh=mesh,
      scratch_types=[
          pltpu.VMEM((b_per_w,), jnp.int32),
          pltpu.VMEM((b_per_w, D), jnp.float32),
          pltpu.VMEM((D, _C), jnp.float32),
          pltpu.VMEM((D, _C), jnp.float32),
          pltpu.SemaphoreType.DMA,
          pltpu.SemaphoreType.DMA,
      ],
      compiler_params=pltpu.CompilerParams(needs_layout_passes=False),
  )
  def gather_kernel(idx_hbm, table_hbm, out_hbm, idx_v, rows_v,
                    stage_a, stage_b, sem_g, sem_s):
    wid = lax.axis_index("s") * info.num_cores + lax.axis_index("c")
    base = wid * b_per_w
    pltpu.sync_copy(idx_hbm.at[pl.ds(base, b_per_w)], idx_v)

    def issue(g, carry):
      v = idx_v[pl.ds(g * 16, 16)]
      for i in range(16):
        pltpu.async_copy(
            table_hbm.at[pl.ds(v[i], 1)], rows_v.at[pl.ds(g * 16 + i, 1)],
            sem_g)
      return carry

    lax.fori_loop(0, b_per_w // 16, issue, 0, unroll=False)

    lanes = lax.iota(jnp.int32, 16)
    cpats = [((c0 + lanes) & 15) for c0 in range(16)]

    writes = [None] * _NCHUNK
    stages = [stage_a, stage_b]
    for ch in range(_NCHUNK):
      pltpu.make_async_copy(
          table_hbm.at[pl.ds(0, _C)], rows_v.at[pl.ds(ch * _C, _C)],
          sem_g).wait()
      if ch - 2 >= 0:
        writes[ch - 2].wait()
      st = stages[ch % 2]

      def transpose_group(g, carry, ch=ch, st=st):
        jvec = ch * _C + g * 16 + lanes
        ivec = g * 16 + lanes
        for cb in range(D // 16):
          for c0 in range(16):
            cvec = cb * 16 + cpats[c0]
            val = plsc.load_gather(rows_v, [jvec, cvec])
            plsc.store_scatter(st, [cvec, ivec], val)
        return carry

      lax.fori_loop(0, _C // 16, transpose_group, 0, unroll=False)
      writes[ch] = pltpu.async_copy(
          st, out_hbm.at[:, pl.ds(base + ch * _C, _C)], sem_s)
    writes[_NCHUNK - 2].wait()
    writes[_NCHUNK - 1].wait()

  return gather_kernel


def kernel(task_ids, table):
  B = task_ids.shape[0]
  V, D = table.shape
  fn = _make_gather(V, D, B)
  return fn(task_ids.astype(jnp.int32), table).T


# speedup vs baseline: 1.2777x; 1.2777x over previous
import functools

import jax
import jax.numpy as jnp
from jax import lax
from jax.experimental import pallas as pl
from jax.experimental.pallas import tpu as pltpu
from jax.experimental.pallas import tpu_sc as plsc

_NCHUNK = 4
_C = 128


def _make_gather(V, D, B):
  info = plsc.get_sparse_core_info()
  NW = info.num_cores * info.num_subcores
  b_per_w = B // NW
  assert b_per_w == _NCHUNK * _C
  mesh = plsc.VectorSubcoreMesh(core_axis_name="c", subcore_axis_name="s")

  @functools.partial(
      pl.kernel,
      out_type=jax.ShapeDtypeStruct((B // 8, 8, D), jnp.float32),
      mesh=mesh,
      scratch_types=[
          pltpu.VMEM((b_per_w,), jnp.int32),
          pltpu.VMEM((b_per_w // 8, 8, D), jnp.float32),
          pltpu.SemaphoreType.DMA,
          pltpu.SemaphoreType.DMA,
      ],
  )
  def gather_kernel(idx_hbm, table_hbm, out_hbm, idx_v, rows_v,
                    sem_g, sem_s):
    wid = lax.axis_index("s") * info.num_cores + lax.axis_index("c")
    base = wid * b_per_w
    pltpu.sync_copy(idx_hbm.at[pl.ds(base, b_per_w)], idx_v)

    def issue(g, carry):
      v = idx_v[pl.ds(g * 16, 16)]
      for i in range(16):
        r = v[i]
        j = g * 16 + i
        pltpu.async_copy(
            table_hbm.at[r >> 3, pl.ds(r & 7, 1)],
            rows_v.at[j >> 3, pl.ds(j & 7, 1)], sem_g)
      return carry

    lax.fori_loop(0, b_per_w // 16, issue, 0, unroll=False)

    writes = []
    for ch in range(_NCHUNK):
      pltpu.make_async_copy(
          table_hbm.at[pl.ds(0, _C // 8)],
          rows_v.at[pl.ds(ch * (_C // 8), _C // 8)], sem_g).wait()
      writes.append(pltpu.async_copy(
          rows_v.at[pl.ds(ch * (_C // 8), _C // 8)],
          out_hbm.at[pl.ds((base + ch * _C) // 8, _C // 8)], sem_s))
    for w in writes:
      w.wait()

  return gather_kernel


def kernel(task_ids, table):
  B = task_ids.shape[0]
  V, D = table.shape
  fn = _make_gather(V, D, B)
  out = fn(task_ids.astype(jnp.int32), table.reshape(V // 8, 8, D))
  return out.reshape(B, D)
